# Initial kernel scaffold; baseline (speedup 1.0000x reference)
import jax
import jax.numpy as jnp
from jax.experimental import pallas as pl


def kernel(x, edge_index, edgenet_input, We1, be1, W0, b0, W1, b1, W2, b2, Wc1, bc1, gamma, beta, Wc2, bc2):
    raise NotImplementedError("write your pallas kernel here")


# speedup vs baseline: 11.6316x; 11.6316x over previous
import functools

import jax
import jax.numpy as jnp
import numpy as np
from jax import lax
from jax.experimental import pallas as pl
from jax.experimental.pallas import tpu as pltpu
from jax.experimental.pallas import tpu_sc as plsc

N = 10000
E = 320000
D = 128
H = 16
EDIM = 16
EH = 128
NC = 2

NUM_TILES = 32
CH = 128
NCH = 80
EPT = CH * NCH
E_PAD = EPT * NUM_TILES
N_PAD = 10240
NPS = N_PAD // 16

_F32 = jnp.float32
_I32 = jnp.int32


_ET = 2560


def _edge_mlp_body(en_ref, w_ref, b_ref, out_ref):
    en = en_ref[...]
    w = w_ref[...]
    b = b_ref[...]
    h1 = jnp.maximum(jnp.dot(en[:, : EDIM // 2], w,
                             preferred_element_type=_F32) + b, 0.0)
    h2 = jnp.maximum(jnp.dot(en[:, EDIM // 2 :], w,
                             preferred_element_type=_F32) + b, 0.0)
    num = jnp.sum(h1 * h2, axis=1, keepdims=True)
    n1 = jnp.sum(h1 * h1, axis=1, keepdims=True)
    n2 = jnp.sum(h2 * h2, axis=1, keepdims=True)
    denom = jnp.maximum(jnp.sqrt(n1 * n2), 1e-8)
    out_ref[...] = (num / denom + 1.0) * 0.5


def _edge_mlp(edgenet_input, We1, be1):
    return pl.pallas_call(
        _edge_mlp_body,
        grid=(E // _ET,),
        in_specs=[
            pl.BlockSpec((_ET, EDIM), lambda i: (i, 0)),
            pl.BlockSpec((EDIM // 2, EH), lambda i: (0, 0)),
            pl.BlockSpec((EH,), lambda i: (0,)),
        ],
        out_specs=pl.BlockSpec((_ET, 1), lambda i: (i, 0)),
        out_shape=jax.ShapeDtypeStruct((E, 1), _F32),
    )(edgenet_input, We1, be1)


@functools.cache
def _sc_mesh():
    return plsc.VectorSubcoreMesh(core_axis_name="c", subcore_axis_name="s")


def _sc_compiler_params():
    return pltpu.CompilerParams(needs_layout_passes=False,
                                use_tc_tiling_on_sc=False)


def _zero_shared_slice(shared, zbuf, sid):
    @pl.loop(0, CH)
    def _(i):
        zbuf[i] = jnp.zeros((16,), _F32)
    base = sid * NPS
    for off in range(0, NPS, CH):
        pltpu.sync_copy(zbuf, shared.at[pl.ds(base + off, CH)])


def _write_out_slice(shared, out_hbm, cid, sid):
    base = sid * NPS
    for off in range(0, NPS, CH):
        pltpu.sync_copy(shared.at[pl.ds(base + off, CH)],
                        out_hbm.at[cid, pl.ds(base + off, CH)])


@functools.cache
def _sc_deg_kernel():
    return pl.kernel(
        _sc_deg_body,
        out_type=jax.ShapeDtypeStruct((2, N_PAD, 16), _F32),
        mesh=_sc_mesh(),
        compiler_params=_sc_compiler_params(),
        scratch_types=[
            pltpu.VMEM((NCH, CH), _I32),
            pltpu.VMEM((NCH, CH), _F32),
            pltpu.VMEM((CH, 16), _F32),
            pltpu.VMEM_SHARED((N_PAD, 16), _F32),
        ],
    )


def _sc_deg(dst_p, ew_p):
    return _sc_deg_kernel()(dst_p, ew_p)


def _sc_deg_body(dst_hbm, ew_hbm, out_hbm, dst_v, ew_v, rows_v, shared):
    cid = lax.axis_index("c")
    sid = lax.axis_index("s")
    wid = cid * 16 + sid
    _zero_shared_slice(shared, rows_v, sid)
    pltpu.sync_copy(dst_hbm.at[wid], dst_v)
    pltpu.sync_copy(ew_hbm.at[wid], ew_v)
    plsc.subcore_barrier()

    @pl.loop(0, NCH)
    def _(j):
        idxj = jnp.zeros((16,), _I32) + j

        @pl.loop(0, CH)
        def _(i):
            idxi = jnp.zeros((16,), _I32) + i
            rows_v[i] = plsc.load_gather(ew_v, [idxj, idxi])

        pltpu.sync_copy(rows_v, shared.at[dst_v.at[j]], add=True)

    plsc.subcore_barrier()
    _write_out_slice(shared, out_hbm, cid, sid)


@functools.cache
def _sc_prop_kernel():
    return pl.kernel(
        _sc_prop_body,
        out_type=jax.ShapeDtypeStruct((2, N_PAD, 16), _F32),
        mesh=_sc_mesh(),
        compiler_params=_sc_compiler_params(),
        scratch_types=[
            pltpu.VMEM((NCH, CH), _I32),
            pltpu.VMEM((NCH, CH), _I32),
            pltpu.VMEM((NCH, CH), _F32),
            pltpu.VMEM((CH, 16), _F32),
            pltpu.VMEM((CH, 16), _F32),
            pltpu.VMEM_SHARED((N_PAD, 16), _F32),
            pltpu.SemaphoreType.DMA,
        ],
    )


def _sc_prop(src_p, dst_p, ew_p, hw):
    return _sc_prop_kernel()(src_p, dst_p, ew_p, hw)


def _sc_prop_body(src_hbm, dst_hbm, ew_hbm, hw_hbm, out_hbm,
                  src_v, dst_v, ew_v, rows_v, zbuf_v, shared, sem):
    cid = lax.axis_index("c")
    sid = lax.axis_index("s")
    wid = cid * 16 + sid
    _zero_shared_slice(shared, zbuf_v, sid)
    pltpu.sync_copy(src_hbm.at[wid], src_v)
    pltpu.sync_copy(dst_hbm.at[wid], dst_v)
    pltpu.sync_copy(ew_hbm.at[wid], ew_v)
    plsc.subcore_barrier()

    @pl.loop(0, NCH)
    def _(j):
        idxj = jnp.zeros((16,), _I32) + j
        pltpu.async_copy(hw_hbm.at[src_v.at[j]], rows_v, sem).wait()

        @pl.loop(0, CH)
        def _(i):
            idxi = jnp.zeros((16,), _I32) + i
            rows_v[i] = rows_v[i] * plsc.load_gather(ew_v, [idxj, idxi])

        pltpu.sync_copy(rows_v, shared.at[dst_v.at[j]], add=True)

    plsc.subcore_barrier()
    _write_out_slice(shared, out_hbm, cid, sid)


_NT = 1000


def _layer0_body(d0_ref, d1_ref, x_ref, w_ref, dinv_ref, hw_ref):
    deg = d0_ref[...] + d1_ref[...] + 1.0
    dinv = lax.rsqrt(deg)
    dinv_ref[...] = dinv
    hw_ref[...] = dinv * jnp.dot(x_ref[...], w_ref[...],
                                 preferred_element_type=_F32)


def _layer0(deg0, deg1, x, W0):
    return pl.pallas_call(
        _layer0_body,
        grid=(N // _NT,),
        in_specs=[
            pl.BlockSpec((_NT, 16), lambda i: (i, 0)),
            pl.BlockSpec((_NT, 16), lambda i: (i, 0)),
            pl.BlockSpec((_NT, D), lambda i: (i, 0)),
            pl.BlockSpec((D, H), lambda i: (0, 0)),
        ],
        out_specs=[
            pl.BlockSpec((_NT, 16), lambda i: (i, 0)),
            pl.BlockSpec((_NT, 16), lambda i: (i, 0)),
        ],
        out_shape=[
            jax.ShapeDtypeStruct((N, 16), _F32),
            jax.ShapeDtypeStruct((N, 16), _F32),
        ],
    )(deg0, deg1, x, W0)


def _layer_body(a0_ref, a1_ref, hwp_ref, dinv_ref, b_ref, w_ref, out_ref):
    dinv = dinv_ref[...]
    h = jnp.maximum(
        dinv * (a0_ref[...] + a1_ref[...] + hwp_ref[...]) + b_ref[...], 0.0)
    out_ref[...] = dinv * jnp.dot(h, w_ref[...], preferred_element_type=_F32)


def _layer(agg0, agg1, hwp, dinv, b, W):
    return pl.pallas_call(
        _layer_body,
        grid=(N // _NT,),
        in_specs=[
            pl.BlockSpec((_NT, 16), lambda i: (i, 0)),
            pl.BlockSpec((_NT, 16), lambda i: (i, 0)),
            pl.BlockSpec((_NT, 16), lambda i: (i, 0)),
            pl.BlockSpec((_NT, 16), lambda i: (i, 0)),
            pl.BlockSpec((H,), lambda i: (0,)),
            pl.BlockSpec((H, H), lambda i: (0, 0)),
        ],
        out_specs=pl.BlockSpec((_NT, 16), lambda i: (i, 0)),
        out_shape=jax.ShapeDtypeStruct((N, 16), _F32),
    )(agg0, agg1, hwp, dinv, b, W)


_BN_SCALE = np.float32(1.0 / np.sqrt(1.0 + 1e-5))


def _head_body(a0_ref, a1_ref, hwp_ref, dinv_ref, b_ref, wc1_ref, bc1_ref,
               g_ref, beta_ref, wc2_ref, bc2_ref, out_ref):
    dinv = dinv_ref[...]
    h = jnp.maximum(
        dinv * (a0_ref[...] + a1_ref[...] + hwp_ref[...]) + b_ref[...], 0.0)
    z = jnp.maximum(
        jnp.dot(h, wc1_ref[...], preferred_element_type=_F32) + bc1_ref[...],
        0.0)
    zbn = z * (g_ref[...] * _BN_SCALE) + beta_ref[...]
    out_ref[...] = jnp.dot(zbn, wc2_ref[...],
                           preferred_element_type=_F32) + bc2_ref[...]


def _head(agg0, agg1, hwp, dinv, b2, Wc1, bc1, gamma, beta, Wc2, bc2):
    return pl.pallas_call(
        _head_body,
        grid=(N // _NT,),
        in_specs=[
            pl.BlockSpec((_NT, 16), lambda i: (i, 0)),
            pl.BlockSpec((_NT, 16), lambda i: (i, 0)),
            pl.BlockSpec((_NT, 16), lambda i: (i, 0)),
            pl.BlockSpec((_NT, 16), lambda i: (i, 0)),
            pl.BlockSpec((H,), lambda i: (0,)),
            pl.BlockSpec((H, 256), lambda i: (0, 0)),
            pl.BlockSpec((256,), lambda i: (0,)),
            pl.BlockSpec((256,), lambda i: (0,)),
            pl.BlockSpec((256,), lambda i: (0,)),
            pl.BlockSpec((256, NC), lambda i: (0, 0)),
            pl.BlockSpec((NC,), lambda i: (0,)),
        ],
        out_specs=pl.BlockSpec((_NT, NC), lambda i: (i, 0)),
        out_shape=jax.ShapeDtypeStruct((N, NC), _F32),
    )(agg0, agg1, hwp, dinv, b2, Wc1, bc1, gamma, beta, Wc2, bc2)


def kernel(x, edge_index, edgenet_input, We1, be1, W0, b0, W1, b1, W2, b2,
           Wc1, bc1, gamma, beta, Wc2, bc2):
    src = edge_index[0].astype(_I32)
    dst = edge_index[1].astype(_I32)

    ew = _edge_mlp(edgenet_input, We1, be1)[:, 0]

    pad = E_PAD - E
    fill = (jnp.arange(pad, dtype=_I32) * 37) % N
    src_p = jnp.concatenate([src, fill]).reshape(NUM_TILES, NCH, CH)
    dst_p = jnp.concatenate([dst, fill]).reshape(NUM_TILES, NCH, CH)
    ew_p = jnp.concatenate([ew, jnp.zeros((pad,), _F32)]
                           ).reshape(NUM_TILES, NCH, CH)

    deg = _sc_deg(dst_p, ew_p)
    dinv, hw = _layer0(deg[0, :N], deg[1, :N], x, W0)

    agg = _sc_prop(src_p, dst_p, ew_p, hw)
    hw1 = _layer(agg[0, :N], agg[1, :N], hw, dinv, b0, W1)
    agg = _sc_prop(src_p, dst_p, ew_p, hw1)
    hw2 = _layer(agg[0, :N], agg[1, :N], hw1, dinv, b1, W2)
    agg = _sc_prop(src_p, dst_p, ew_p, hw2)

    return _head(agg[0, :N], agg[1, :N], hw2, dinv, b2, Wc1, bc1, gamma,
                 beta, Wc2, bc2)


# speedup vs baseline: 16.5251x; 1.4207x over previous
import functools

import jax
import jax.numpy as jnp
import numpy as np
from jax import lax
from jax.experimental import pallas as pl
from jax.experimental.pallas import tpu as pltpu
from jax.experimental.pallas import tpu_sc as plsc

N = 10000
E = 320000
D = 128
H = 16
EDIM = 16
EH = 128
NC = 2

NUM_TILES = 32
CH = 128
NCH = 80
EPT = CH * NCH
E_PAD = EPT * NUM_TILES
N_PAD = 10240
NPS = N_PAD // 16

_F32 = jnp.float32
_I32 = jnp.int32


_ET = 2560


def _edge_mlp_body(en_ref, w_ref, b_ref, out_ref):
    en = en_ref[...]
    w = w_ref[...]
    b = b_ref[...]
    h1 = jnp.maximum(jnp.dot(en[:, : EDIM // 2], w,
                             preferred_element_type=_F32) + b, 0.0)
    h2 = jnp.maximum(jnp.dot(en[:, EDIM // 2 :], w,
                             preferred_element_type=_F32) + b, 0.0)
    num = jnp.sum(h1 * h2, axis=1, keepdims=True)
    n1 = jnp.sum(h1 * h1, axis=1, keepdims=True)
    n2 = jnp.sum(h2 * h2, axis=1, keepdims=True)
    denom = jnp.maximum(jnp.sqrt(n1 * n2), 1e-8)
    out_ref[...] = (num / denom + 1.0) * 0.5


def _edge_mlp(edgenet_input, We1, be1):
    return pl.pallas_call(
        _edge_mlp_body,
        grid=(E // _ET,),
        in_specs=[
            pl.BlockSpec((_ET, EDIM), lambda i: (i, 0)),
            pl.BlockSpec((EDIM // 2, EH), lambda i: (0, 0)),
            pl.BlockSpec((EH,), lambda i: (0,)),
        ],
        out_specs=pl.BlockSpec((_ET, 1), lambda i: (i, 0)),
        out_shape=jax.ShapeDtypeStruct((E, 1), _F32),
    )(edgenet_input, We1, be1)


@functools.cache
def _sc_mesh():
    return plsc.VectorSubcoreMesh(core_axis_name="c", subcore_axis_name="s")


def _sc_compiler_params():
    return pltpu.CompilerParams(needs_layout_passes=False,
                                use_tc_tiling_on_sc=False)


def _splat(vec16, k):
    idx = jnp.full((16,), k, _I32)
    return vec16.at[idx].get(mode="promise_in_bounds")


def _zero_shared_slice(shared, zbuf, sid):
    @pl.loop(0, CH)
    def _(i):
        zbuf[i] = jnp.zeros((16,), _F32)
    base = sid * NPS
    for off in range(0, NPS, CH):
        pltpu.sync_copy(zbuf, shared.at[pl.ds(base + off, CH)])


def _write_out_slice(shared, out_hbm, cid, sid):
    base = sid * NPS
    for off in range(0, NPS, CH):
        pltpu.sync_copy(shared.at[pl.ds(base + off, CH)],
                        out_hbm.at[cid, pl.ds(base + off, CH)])


@functools.cache
def _sc_deg_kernel():
    return pl.kernel(
        _sc_deg_body,
        out_type=jax.ShapeDtypeStruct((2, N_PAD, 16), _F32),
        mesh=_sc_mesh(),
        compiler_params=_sc_compiler_params(),
        scratch_types=[
            pltpu.VMEM((NCH, CH), _I32),
            pltpu.VMEM((NCH, CH), _F32),
            pltpu.VMEM((CH, 16), _F32),
            pltpu.VMEM_SHARED((N_PAD, 16), _F32),
        ],
    )


def _sc_deg(dst_p, ew_p):
    return _sc_deg_kernel()(dst_p, ew_p)


def _sc_deg_body(dst_hbm, ew_hbm, out_hbm, dst_v, ew_v, rows_v, shared):
    cid = lax.axis_index("c")
    sid = lax.axis_index("s")
    wid = cid * 16 + sid
    _zero_shared_slice(shared, rows_v, sid)
    pltpu.sync_copy(dst_hbm.at[wid], dst_v)
    pltpu.sync_copy(ew_hbm.at[wid], ew_v)
    plsc.subcore_barrier()

    @pl.loop(0, NCH)
    def _(j):
        for g in range(CH // 16):
            ew16 = ew_v[j, pl.ds(g * 16, 16)]
            for k in range(16):
                rows_v[g * 16 + k] = _splat(ew16, k)

        pltpu.sync_copy(rows_v, shared.at[dst_v.at[j]], add=True)

    plsc.subcore_barrier()
    _write_out_slice(shared, out_hbm, cid, sid)


@functools.cache
def _sc_prop_kernel():
    return pl.kernel(
        _sc_prop_body,
        out_type=jax.ShapeDtypeStruct((2, N_PAD, 16), _F32),
        mesh=_sc_mesh(),
        compiler_params=_sc_compiler_params(),
        scratch_types=[
            pltpu.VMEM((NCH, CH), _I32),
            pltpu.VMEM((NCH, CH), _I32),
            pltpu.VMEM((NCH, CH), _F32),
            pltpu.VMEM((CH, 16), _F32),
            pltpu.VMEM((CH, 16), _F32),
            pltpu.VMEM((CH, 16), _F32),
            pltpu.VMEM_SHARED((N_PAD, 16), _F32),
            pltpu.SemaphoreType.DMA,
            pltpu.SemaphoreType.DMA,
            pltpu.SemaphoreType.DMA,
            pltpu.SemaphoreType.DMA,
        ],
    )


def _sc_prop(src_p, dst_p, ew_p, hw):
    return _sc_prop_kernel()(src_p, dst_p, ew_p, hw)


def _sc_prop_body(src_hbm, dst_hbm, ew_hbm, hw_hbm, out_hbm,
                  src_v, dst_v, ew_v, rows_a, rows_b, zbuf_v, shared,
                  gsem_a, gsem_b, ssem_a, ssem_b):
    cid = lax.axis_index("c")
    sid = lax.axis_index("s")
    wid = cid * 16 + sid
    _zero_shared_slice(shared, zbuf_v, sid)
    pltpu.sync_copy(src_hbm.at[wid], src_v)
    pltpu.sync_copy(dst_hbm.at[wid], dst_v)
    pltpu.sync_copy(ew_hbm.at[wid], ew_v)
    plsc.subcore_barrier()

    bufs = ((rows_a, gsem_a, ssem_a), (rows_b, gsem_b, ssem_b))

    def _scale(rows_v, j):
        for g in range(CH // 16):
            ew16 = ew_v[j, pl.ds(g * 16, 16)]
            for k in range(16):
                i = g * 16 + k
                rows_v[i] = rows_v[i] * _splat(ew16, k)

    def _gather_start(b, j):
        rows_v, gsem, _ = bufs[b]
        pltpu.async_copy(hw_hbm.at[src_v.at[j]], rows_v, gsem)

    def _process(b, j):
        rows_v, gsem, ssem = bufs[b]
        pltpu.make_async_copy(hw_hbm.at[src_v.at[j]], rows_v, gsem).wait()
        _scale(rows_v, j)
        pltpu.async_copy(rows_v, shared.at[dst_v.at[j]], ssem, add=True)

    def _refill(b, jnext):
        rows_v, gsem, ssem = bufs[b]
        pltpu.make_async_copy(rows_v, shared.at[dst_v.at[jnext]], ssem).wait()
        _gather_start(b, jnext)

    _gather_start(0, 0)
    _gather_start(1, 1)

    @pl.loop(0, NCH // 2 - 1)
    def _(p):
        j = p * 2
        _process(0, j)
        _refill(0, j + 2)
        _process(1, j + 1)
        _refill(1, j + 3)

    _process(0, NCH - 2)
    _process(1, NCH - 1)
    pltpu.make_async_copy(rows_a, shared.at[dst_v.at[NCH - 2]], ssem_a).wait()
    pltpu.make_async_copy(rows_b, shared.at[dst_v.at[NCH - 1]], ssem_b).wait()

    plsc.subcore_barrier()
    _write_out_slice(shared, out_hbm, cid, sid)


_NT = 1000


def _layer0_body(d0_ref, d1_ref, x_ref, w_ref, dinv_ref, hw_ref):
    deg = d0_ref[...] + d1_ref[...] + 1.0
    dinv = lax.rsqrt(deg)
    dinv_ref[...] = dinv
    hw_ref[...] = dinv * jnp.dot(x_ref[...], w_ref[...],
                                 preferred_element_type=_F32)


def _layer0(deg0, deg1, x, W0):
    return pl.pallas_call(
        _layer0_body,
        grid=(N // _NT,),
        in_specs=[
            pl.BlockSpec((_NT, 16), lambda i: (i, 0)),
            pl.BlockSpec((_NT, 16), lambda i: (i, 0)),
            pl.BlockSpec((_NT, D), lambda i: (i, 0)),
            pl.BlockSpec((D, H), lambda i: (0, 0)),
        ],
        out_specs=[
            pl.BlockSpec((_NT, 16), lambda i: (i, 0)),
            pl.BlockSpec((_NT, 16), lambda i: (i, 0)),
        ],
        out_shape=[
            jax.ShapeDtypeStruct((N, 16), _F32),
            jax.ShapeDtypeStruct((N, 16), _F32),
        ],
    )(deg0, deg1, x, W0)


def _layer_body(a0_ref, a1_ref, hwp_ref, dinv_ref, b_ref, w_ref, out_ref):
    dinv = dinv_ref[...]
    h = jnp.maximum(
        dinv * (a0_ref[...] + a1_ref[...] + hwp_ref[...]) + b_ref[...], 0.0)
    out_ref[...] = dinv * jnp.dot(h, w_ref[...], preferred_element_type=_F32)


def _layer(agg0, agg1, hwp, dinv, b, W):
    return pl.pallas_call(
        _layer_body,
        grid=(N // _NT,),
        in_specs=[
            pl.BlockSpec((_NT, 16), lambda i: (i, 0)),
            pl.BlockSpec((_NT, 16), lambda i: (i, 0)),
            pl.BlockSpec((_NT, 16), lambda i: (i, 0)),
            pl.BlockSpec((_NT, 16), lambda i: (i, 0)),
            pl.BlockSpec((H,), lambda i: (0,)),
            pl.BlockSpec((H, H), lambda i: (0, 0)),
        ],
        out_specs=pl.BlockSpec((_NT, 16), lambda i: (i, 0)),
        out_shape=jax.ShapeDtypeStruct((N, 16), _F32),
    )(agg0, agg1, hwp, dinv, b, W)


_BN_SCALE = np.float32(1.0 / np.sqrt(1.0 + 1e-5))


def _head_body(a0_ref, a1_ref, hwp_ref, dinv_ref, b_ref, wc1_ref, bc1_ref,
               g_ref, beta_ref, wc2_ref, bc2_ref, out_ref):
    dinv = dinv_ref[...]
    h = jnp.maximum(
        dinv * (a0_ref[...] + a1_ref[...] + hwp_ref[...]) + b_ref[...], 0.0)
    z = jnp.maximum(
        jnp.dot(h, wc1_ref[...], preferred_element_type=_F32) + bc1_ref[...],
        0.0)
    zbn = z * (g_ref[...] * _BN_SCALE) + beta_ref[...]
    out_ref[...] = jnp.dot(zbn, wc2_ref[...],
                           preferred_element_type=_F32) + bc2_ref[...]


def _head(agg0, agg1, hwp, dinv, b2, Wc1, bc1, gamma, beta, Wc2, bc2):
    return pl.pallas_call(
        _head_body,
        grid=(N // _NT,),
        in_specs=[
            pl.BlockSpec((_NT, 16), lambda i: (i, 0)),
            pl.BlockSpec((_NT, 16), lambda i: (i, 0)),
            pl.BlockSpec((_NT, 16), lambda i: (i, 0)),
            pl.BlockSpec((_NT, 16), lambda i: (i, 0)),
            pl.BlockSpec((H,), lambda i: (0,)),
            pl.BlockSpec((H, 256), lambda i: (0, 0)),
            pl.BlockSpec((256,), lambda i: (0,)),
            pl.BlockSpec((256,), lambda i: (0,)),
            pl.BlockSpec((256,), lambda i: (0,)),
            pl.BlockSpec((256, NC), lambda i: (0, 0)),
            pl.BlockSpec((NC,), lambda i: (0,)),
        ],
        out_specs=pl.BlockSpec((_NT, NC), lambda i: (i, 0)),
        out_shape=jax.ShapeDtypeStruct((N, NC), _F32),
    )(agg0, agg1, hwp, dinv, b2, Wc1, bc1, gamma, beta, Wc2, bc2)


def kernel(x, edge_index, edgenet_input, We1, be1, W0, b0, W1, b1, W2, b2,
           Wc1, bc1, gamma, beta, Wc2, bc2):
    src = edge_index[0].astype(_I32)
    dst = edge_index[1].astype(_I32)

    ew = _edge_mlp(edgenet_input, We1, be1)[:, 0]

    pad = E_PAD - E
    fill = (jnp.arange(pad, dtype=_I32) * 37) % N
    src_p = jnp.concatenate([src, fill]).reshape(NUM_TILES, NCH, CH)
    dst_p = jnp.concatenate([dst, fill]).reshape(NUM_TILES, NCH, CH)
    ew_p = jnp.concatenate([ew, jnp.zeros((pad,), _F32)]
                           ).reshape(NUM_TILES, NCH, CH)

    deg = _sc_deg(dst_p, ew_p)
    dinv, hw = _layer0(deg[0, :N], deg[1, :N], x, W0)

    agg = _sc_prop(src_p, dst_p, ew_p, hw)
    hw1 = _layer(agg[0, :N], agg[1, :N], hw, dinv, b0, W1)
    agg = _sc_prop(src_p, dst_p, ew_p, hw1)
    hw2 = _layer(agg[0, :N], agg[1, :N], hw1, dinv, b1, W2)
    agg = _sc_prop(src_p, dst_p, ew_p, hw2)

    return _head(agg[0, :N], agg[1, :N], hw2, dinv, b2, Wc1, bc1, gamma,
                 beta, Wc2, bc2)


# speedup vs baseline: 26.3472x; 1.5944x over previous
import functools

import jax
import jax.numpy as jnp
import numpy as np
from jax import lax
from jax.experimental import pallas as pl
from jax.experimental.pallas import tpu as pltpu
from jax.experimental.pallas import tpu_sc as plsc

N = 10000
E = 320000
D = 128
H = 16
EDIM = 16
EH = 128
NC = 2

NUM_TILES = 32
CH = 128
NCH = 80
EPT = CH * NCH
E_PAD = EPT * NUM_TILES
N_PAD = 10240
NPS = N_PAD // 16

_F32 = jnp.float32
_I32 = jnp.int32


_ET = 2560


def _edge_mlp_body(ent_ref, w_ref, b_ref, out_ref):
    ent = ent_ref[...]
    wt = w_ref[...].T
    b = b_ref[...][:, None]
    h1 = jnp.maximum(jnp.dot(wt, ent[: EDIM // 2, :],
                             preferred_element_type=_F32) + b, 0.0)
    h2 = jnp.maximum(jnp.dot(wt, ent[EDIM // 2 :, :],
                             preferred_element_type=_F32) + b, 0.0)
    num = jnp.sum(h1 * h2, axis=0, keepdims=True)
    n1 = jnp.sum(h1 * h1, axis=0, keepdims=True)
    n2 = jnp.sum(h2 * h2, axis=0, keepdims=True)
    denom = jnp.maximum(jnp.sqrt(n1 * n2), 1e-8)
    ew = (num / denom + 1.0) * 0.5
    out_ref[...] = ew.reshape(1, _ET // 128, 128)


def _edge_mlp(edgenet_input, We1, be1):
    ent = edgenet_input.T
    return pl.pallas_call(
        _edge_mlp_body,
        grid=(E // _ET,),
        in_specs=[
            pl.BlockSpec((EDIM, _ET), lambda i: (0, i)),
            pl.BlockSpec((EDIM // 2, EH), lambda i: (0, 0)),
            pl.BlockSpec((EH,), lambda i: (0,)),
        ],
        out_specs=pl.BlockSpec((1, _ET // 128, 128), lambda i: (i, 0, 0)),
        out_shape=jax.ShapeDtypeStruct((E // _ET, _ET // 128, 128), _F32),
    )(ent, We1, be1)


@functools.cache
def _sc_mesh():
    return plsc.VectorSubcoreMesh(core_axis_name="c", subcore_axis_name="s")


def _sc_compiler_params():
    return pltpu.CompilerParams(needs_layout_passes=False,
                                use_tc_tiling_on_sc=False)


def _splat(vec16, k):
    idx = jnp.full((16,), k, _I32)
    return vec16.at[idx].get(mode="promise_in_bounds")


def _zero_shared_slice(shared, zbuf, sid):
    @pl.loop(0, CH)
    def _(i):
        zbuf[i] = jnp.zeros((16,), _F32)
    base = sid * NPS
    for off in range(0, NPS, CH):
        pltpu.sync_copy(zbuf, shared.at[pl.ds(base + off, CH)])


def _write_out_slice(shared, out_hbm, cid, sid):
    base = sid * NPS
    for off in range(0, NPS, CH):
        pltpu.sync_copy(shared.at[pl.ds(base + off, CH)],
                        out_hbm.at[cid, pl.ds(base + off, CH)])


@functools.cache
def _sc_deg_kernel():
    return pl.kernel(
        _sc_deg_body,
        out_type=jax.ShapeDtypeStruct((2, N_PAD, 16), _F32),
        mesh=_sc_mesh(),
        compiler_params=_sc_compiler_params(),
        scratch_types=[
            pltpu.VMEM((NCH, CH), _I32),
            pltpu.VMEM((NCH, CH), _F32),
            pltpu.VMEM((CH, 16), _F32),
            pltpu.VMEM((CH, 16), _F32),
            pltpu.VMEM_SHARED((N_PAD, 16), _F32),
            pltpu.SemaphoreType.DMA,
            pltpu.SemaphoreType.DMA,
        ],
    )


def _sc_deg(dst_p, ew_p):
    return _sc_deg_kernel()(dst_p, ew_p)


def _sc_deg_body(dst_hbm, ew_hbm, out_hbm, dst_v, ew_v, rows_a, rows_b,
                 shared, ssem_a, ssem_b):
    cid = lax.axis_index("c")
    sid = lax.axis_index("s")
    wid = cid * 16 + sid
    _zero_shared_slice(shared, rows_a, sid)
    pltpu.sync_copy(dst_hbm.at[wid], dst_v)
    pltpu.sync_copy(ew_hbm.at[wid], ew_v)
    plsc.subcore_barrier()

    bufs = ((rows_a, ssem_a), (rows_b, ssem_b))

    def _fill(rows_v, j):
        for g in range(CH // 16):
            ew16 = ew_v[j, pl.ds(g * 16, 16)]
            for k in range(16):
                rows_v[g * 16 + k] = _splat(ew16, k)

    def _step(b, j, drain):
        rows_v, ssem = bufs[b]
        if drain:
            pltpu.make_async_copy(rows_v, shared.at[dst_v.at[j]], ssem).wait()
        _fill(rows_v, j)
        pltpu.async_copy(rows_v, shared.at[dst_v.at[j]], ssem, add=True)

    _step(0, 0, False)
    _step(1, 1, False)

    @pl.loop(1, NCH // 2)
    def _(p):
        _step(0, p * 2, True)
        _step(1, p * 2 + 1, True)

    pltpu.make_async_copy(rows_a, shared.at[dst_v.at[0]], ssem_a).wait()
    pltpu.make_async_copy(rows_b, shared.at[dst_v.at[0]], ssem_b).wait()

    plsc.subcore_barrier()
    _write_out_slice(shared, out_hbm, cid, sid)


@functools.cache
def _sc_prop_kernel():
    return pl.kernel(
        _sc_prop_body,
        out_type=jax.ShapeDtypeStruct((2, N_PAD, 16), _F32),
        mesh=_sc_mesh(),
        compiler_params=_sc_compiler_params(),
        scratch_types=[
            pltpu.VMEM((NCH, CH), _I32),
            pltpu.VMEM((NCH, CH), _I32),
            pltpu.VMEM((NCH, CH), _F32),
            pltpu.VMEM((CH, 16), _F32),
            pltpu.VMEM((CH, 16), _F32),
            pltpu.VMEM((CH, 16), _F32),
            pltpu.VMEM_SHARED((N_PAD, 16), _F32),
            pltpu.SemaphoreType.DMA,
            pltpu.SemaphoreType.DMA,
            pltpu.SemaphoreType.DMA,
            pltpu.SemaphoreType.DMA,
        ],
    )


def _sc_prop(src_p, dst_p, ew_p, hw):
    return _sc_prop_kernel()(src_p, dst_p, ew_p, hw)


def _sc_prop_body(src_hbm, dst_hbm, ew_hbm, hw_hbm, out_hbm,
                  src_v, dst_v, ew_v, rows_a, rows_b, zbuf_v, shared,
                  gsem_a, gsem_b, ssem_a, ssem_b):
    cid = lax.axis_index("c")
    sid = lax.axis_index("s")
    wid = cid * 16 + sid
    _zero_shared_slice(shared, zbuf_v, sid)
    pltpu.sync_copy(src_hbm.at[wid], src_v)
    pltpu.sync_copy(dst_hbm.at[wid], dst_v)
    pltpu.sync_copy(ew_hbm.at[wid], ew_v)
    plsc.subcore_barrier()

    bufs = ((rows_a, gsem_a, ssem_a), (rows_b, gsem_b, ssem_b))

    def _scale(rows_v, j):
        for g in range(CH // 16):
            ew16 = ew_v[j, pl.ds(g * 16, 16)]
            for k in range(16):
                i = g * 16 + k
                rows_v[i] = rows_v[i] * _splat(ew16, k)

    def _gather_start(b, j):
        rows_v, gsem, _ = bufs[b]
        pltpu.async_copy(hw_hbm.at[src_v.at[j]], rows_v, gsem)

    def _process(b, j):
        rows_v, gsem, ssem = bufs[b]
        pltpu.make_async_copy(hw_hbm.at[src_v.at[j]], rows_v, gsem).wait()
        _scale(rows_v, j)
        pltpu.async_copy(rows_v, shared.at[dst_v.at[j]], ssem, add=True)

    def _refill(b, jnext):
        rows_v, gsem, ssem = bufs[b]
        pltpu.make_async_copy(rows_v, shared.at[dst_v.at[jnext]], ssem).wait()
        _gather_start(b, jnext)

    _gather_start(0, 0)
    _gather_start(1, 1)

    @pl.loop(0, NCH // 2 - 1)
    def _(p):
        j = p * 2
        _process(0, j)
        _refill(0, j + 2)
        _process(1, j + 1)
        _refill(1, j + 3)

    _process(0, NCH - 2)
    _process(1, NCH - 1)
    pltpu.make_async_copy(rows_a, shared.at[dst_v.at[NCH - 2]], ssem_a).wait()
    pltpu.make_async_copy(rows_b, shared.at[dst_v.at[NCH - 1]], ssem_b).wait()

    plsc.subcore_barrier()
    _write_out_slice(shared, out_hbm, cid, sid)


_NT = 1280


def _xw0_body(x_ref, w_ref, out_ref):
    out_ref[...] = jnp.dot(x_ref[...], w_ref[...],
                           preferred_element_type=_F32)


def _xw0(x, W0):
    return pl.pallas_call(
        _xw0_body,
        grid=(N_PAD // _NT,),
        in_specs=[
            pl.BlockSpec((_NT, D), lambda i: (i, 0)),
            pl.BlockSpec((D, H), lambda i: (0, 0)),
        ],
        out_specs=pl.BlockSpec((_NT, 16), lambda i: (i, 0)),
        out_shape=jax.ShapeDtypeStruct((N_PAD, 16), _F32),
    )(x, W0)


def _layer0_body(d0_ref, d1_ref, xw_ref, dinv_ref, hw_ref):
    deg = d0_ref[...] + d1_ref[...] + 1.0
    dinv = lax.rsqrt(deg)
    dinv_ref[...] = dinv
    hw_ref[...] = dinv * xw_ref[...]


def _layer0(deg0, deg1, xw):
    return pl.pallas_call(
        _layer0_body,
        grid=(N_PAD // _NT,),
        in_specs=[
            pl.BlockSpec((_NT, 16), lambda i: (i, 0)),
            pl.BlockSpec((_NT, 16), lambda i: (i, 0)),
            pl.BlockSpec((_NT, 16), lambda i: (i, 0)),
        ],
        out_specs=[
            pl.BlockSpec((_NT, 16), lambda i: (i, 0)),
            pl.BlockSpec((_NT, 16), lambda i: (i, 0)),
        ],
        out_shape=[
            jax.ShapeDtypeStruct((N_PAD, 16), _F32),
            jax.ShapeDtypeStruct((N_PAD, 16), _F32),
        ],
    )(deg0, deg1, xw)


def _layer_body(a0_ref, a1_ref, hwp_ref, dinv_ref, b_ref, w_ref, out_ref):
    dinv = dinv_ref[...]
    h = jnp.maximum(
        dinv * (a0_ref[...] + a1_ref[...] + hwp_ref[...]) + b_ref[...], 0.0)
    out_ref[...] = dinv * jnp.dot(h, w_ref[...], preferred_element_type=_F32)


def _layer(agg0, agg1, hwp, dinv, b, W):
    return pl.pallas_call(
        _layer_body,
        grid=(N_PAD // _NT,),
        in_specs=[
            pl.BlockSpec((_NT, 16), lambda i: (i, 0)),
            pl.BlockSpec((_NT, 16), lambda i: (i, 0)),
            pl.BlockSpec((_NT, 16), lambda i: (i, 0)),
            pl.BlockSpec((_NT, 16), lambda i: (i, 0)),
            pl.BlockSpec((H,), lambda i: (0,)),
            pl.BlockSpec((H, H), lambda i: (0, 0)),
        ],
        out_specs=pl.BlockSpec((_NT, 16), lambda i: (i, 0)),
        out_shape=jax.ShapeDtypeStruct((N_PAD, 16), _F32),
    )(agg0, agg1, hwp, dinv, b, W)


_BN_SCALE = np.float32(1.0 / np.sqrt(1.0 + 1e-5))


def _head_body(a0_ref, a1_ref, hwp_ref, dinv_ref, b_ref, wc1_ref, bc1_ref,
               g_ref, beta_ref, wc2_ref, bc2_ref, out_ref):
    dinv = dinv_ref[...]
    h = jnp.maximum(
        dinv * (a0_ref[...] + a1_ref[...] + hwp_ref[...]) + b_ref[...], 0.0)
    z = jnp.maximum(
        jnp.dot(h, wc1_ref[...], preferred_element_type=_F32) + bc1_ref[...],
        0.0)
    zbn = z * (g_ref[...] * _BN_SCALE) + beta_ref[...]
    out_ref[...] = jnp.dot(zbn, wc2_ref[...],
                           preferred_element_type=_F32) + bc2_ref[...]


def _head(agg0, agg1, hwp, dinv, b2, Wc1, bc1, gamma, beta, Wc2, bc2):
    return pl.pallas_call(
        _head_body,
        grid=(N_PAD // _NT,),
        in_specs=[
            pl.BlockSpec((_NT, 16), lambda i: (i, 0)),
            pl.BlockSpec((_NT, 16), lambda i: (i, 0)),
            pl.BlockSpec((_NT, 16), lambda i: (i, 0)),
            pl.BlockSpec((_NT, 16), lambda i: (i, 0)),
            pl.BlockSpec((H,), lambda i: (0,)),
            pl.BlockSpec((H, 256), lambda i: (0, 0)),
            pl.BlockSpec((256,), lambda i: (0,)),
            pl.BlockSpec((256,), lambda i: (0,)),
            pl.BlockSpec((256,), lambda i: (0,)),
            pl.BlockSpec((256, NC), lambda i: (0, 0)),
            pl.BlockSpec((NC,), lambda i: (0,)),
        ],
        out_specs=pl.BlockSpec((_NT, NC), lambda i: (i, 0)),
        out_shape=jax.ShapeDtypeStruct((N_PAD, NC), _F32),
    )(agg0, agg1, hwp, dinv, b2, Wc1, bc1, gamma, beta, Wc2, bc2)


def kernel(x, edge_index, edgenet_input, We1, be1, W0, b0, W1, b1, W2, b2,
           Wc1, bc1, gamma, beta, Wc2, bc2):
    src = edge_index[0].astype(_I32)
    dst = edge_index[1].astype(_I32)

    ew2d = _edge_mlp(edgenet_input, We1, be1).reshape(E // 128, 128)

    pad = E_PAD - E
    fill = (jnp.arange(pad, dtype=_I32) * 37) % N
    src_p = jnp.concatenate([src, fill]).reshape(NUM_TILES, NCH, CH)
    dst_p = jnp.concatenate([dst, fill]).reshape(NUM_TILES, NCH, CH)
    ew_p = jnp.concatenate([ew2d, jnp.zeros((pad // 128, 128), _F32)]
                           ).reshape(NUM_TILES, NCH, CH)

    x_pad = jnp.concatenate([x, jnp.zeros((N_PAD - N, D), _F32)])
    xw = _xw0(x_pad, W0)

    deg = _sc_deg(dst_p, ew_p)
    dinv, hw = _layer0(deg[0], deg[1], xw)

    agg = _sc_prop(src_p, dst_p, ew_p, hw)
    hw1 = _layer(agg[0], agg[1], hw, dinv, b0, W1)
    agg = _sc_prop(src_p, dst_p, ew_p, hw1)
    hw2 = _layer(agg[0], agg[1], hw1, dinv, b1, W2)
    agg = _sc_prop(src_p, dst_p, ew_p, hw2)

    logit = _head(agg[0], agg[1], hw2, dinv, b2, Wc1, bc1, gamma,
                  beta, Wc2, bc2)
    return logit[:N]


# speedup vs baseline: 28.5447x; 1.0834x over previous
import functools

import jax
import jax.numpy as jnp
import numpy as np
from jax import lax
from jax.experimental import pallas as pl
from jax.experimental.pallas import tpu as pltpu
from jax.experimental.pallas import tpu_sc as plsc

N = 10000
E = 320000
D = 128
H = 16
EDIM = 16
EH = 128
NC = 2

NUM_TILES = 32
CH = 128
NCH = 80
EPT = CH * NCH
E_PAD = EPT * NUM_TILES
N_PAD = 10240
NPS = N_PAD // 16

_F32 = jnp.float32
_I32 = jnp.int32


_ET = 2560


def _edge_mlp_body(ent_ref, w_ref, b_ref, out_ref):
    ent = ent_ref[...]
    wt = w_ref[...].T
    b = b_ref[...][:, None]
    h1 = jnp.maximum(jnp.dot(wt, ent[: EDIM // 2, :],
                             preferred_element_type=_F32,
                             precision=lax.Precision.DEFAULT) + b, 0.0)
    h2 = jnp.maximum(jnp.dot(wt, ent[EDIM // 2 :, :],
                             preferred_element_type=_F32,
                             precision=lax.Precision.DEFAULT) + b, 0.0)
    num = jnp.sum(h1 * h2, axis=0, keepdims=True)
    n1 = jnp.sum(h1 * h1, axis=0, keepdims=True)
    n2 = jnp.sum(h2 * h2, axis=0, keepdims=True)
    denom = jnp.maximum(jnp.sqrt(n1 * n2), 1e-8)
    ew = (num / denom + 1.0) * 0.5
    out_ref[...] = ew.reshape(1, _ET // 128, 128)


def _edge_mlp(edgenet_input, We1, be1):
    ent = edgenet_input.T
    return pl.pallas_call(
        _edge_mlp_body,
        grid=(E // _ET,),
        in_specs=[
            pl.BlockSpec((EDIM, _ET), lambda i: (0, i)),
            pl.BlockSpec((EDIM // 2, EH), lambda i: (0, 0)),
            pl.BlockSpec((EH,), lambda i: (0,)),
        ],
        out_specs=pl.BlockSpec((1, _ET // 128, 128), lambda i: (i, 0, 0)),
        out_shape=jax.ShapeDtypeStruct((E // _ET, _ET // 128, 128), _F32),
    )(ent, We1, be1)


@functools.cache
def _sc_mesh():
    return plsc.VectorSubcoreMesh(core_axis_name="c", subcore_axis_name="s")


def _sc_compiler_params():
    return pltpu.CompilerParams(needs_layout_passes=False,
                                use_tc_tiling_on_sc=False)


def _splat(vec16, k):
    idx = jnp.full((16,), k, _I32)
    return vec16.at[idx].get(mode="promise_in_bounds")


def _zero_shared_slice(shared, zbuf, sid):
    @pl.loop(0, CH)
    def _(i):
        zbuf[i] = jnp.zeros((16,), _F32)
    base = sid * NPS
    for off in range(0, NPS, CH):
        pltpu.sync_copy(zbuf, shared.at[pl.ds(base + off, CH)])


def _write_out_slice(shared, out_hbm, cid, sid):
    base = sid * NPS
    for off in range(0, NPS, CH):
        pltpu.sync_copy(shared.at[pl.ds(base + off, CH)],
                        out_hbm.at[cid, pl.ds(base + off, CH)])


@functools.cache
def _sc_deg_kernel():
    return pl.kernel(
        _sc_deg_body,
        out_type=jax.ShapeDtypeStruct((2, N_PAD, 16), _F32),
        mesh=_sc_mesh(),
        compiler_params=_sc_compiler_params(),
        scratch_types=[
            pltpu.VMEM((NCH, CH), _I32),
            pltpu.VMEM((NCH, CH), _F32),
            pltpu.VMEM((CH, 16), _F32),
            pltpu.VMEM((CH, 16), _F32),
            pltpu.VMEM_SHARED((N_PAD, 16), _F32),
            pltpu.SemaphoreType.DMA,
            pltpu.SemaphoreType.DMA,
        ],
    )


def _sc_deg(dst_p, ew_p):
    return _sc_deg_kernel()(dst_p, ew_p)


def _sc_deg_body(dst_hbm, ew_hbm, out_hbm, dst_v, ew_v, rows_a, rows_b,
                 shared, ssem_a, ssem_b):
    cid = lax.axis_index("c")
    sid = lax.axis_index("s")
    wid = cid * 16 + sid
    _zero_shared_slice(shared, rows_a, sid)
    pltpu.sync_copy(dst_hbm.at[wid], dst_v)
    pltpu.sync_copy(ew_hbm.at[wid], ew_v)
    plsc.subcore_barrier()

    bufs = ((rows_a, ssem_a), (rows_b, ssem_b))

    def _fill(rows_v, j):
        for g in range(CH // 16):
            ew16 = ew_v[j, pl.ds(g * 16, 16)]
            for k in range(16):
                rows_v[g * 16 + k] = _splat(ew16, k)

    def _step(b, j, drain):
        rows_v, ssem = bufs[b]
        if drain:
            pltpu.make_async_copy(rows_v, shared.at[dst_v.at[j]], ssem).wait()
        _fill(rows_v, j)
        pltpu.async_copy(rows_v, shared.at[dst_v.at[j]], ssem, add=True)

    _step(0, 0, False)
    _step(1, 1, False)

    @pl.loop(1, NCH // 2)
    def _(p):
        _step(0, p * 2, True)
        _step(1, p * 2 + 1, True)

    pltpu.make_async_copy(rows_a, shared.at[dst_v.at[0]], ssem_a).wait()
    pltpu.make_async_copy(rows_b, shared.at[dst_v.at[0]], ssem_b).wait()

    plsc.subcore_barrier()
    _write_out_slice(shared, out_hbm, cid, sid)


@functools.cache
def _sc_prop_kernel():
    return pl.kernel(
        _sc_prop_body,
        out_type=jax.ShapeDtypeStruct((2, N_PAD, 16), _F32),
        mesh=_sc_mesh(),
        compiler_params=_sc_compiler_params(),
        scratch_types=[
            pltpu.VMEM((NCH, CH), _I32),
            pltpu.VMEM((NCH, CH), _I32),
            pltpu.VMEM((NCH, CH), _F32),
            pltpu.VMEM((CH, 16), _F32),
            pltpu.VMEM((CH, 16), _F32),
            pltpu.VMEM((CH, 16), _F32),
            pltpu.VMEM_SHARED((N_PAD, 16), _F32),
            pltpu.SemaphoreType.DMA,
            pltpu.SemaphoreType.DMA,
            pltpu.SemaphoreType.DMA,
            pltpu.SemaphoreType.DMA,
        ],
    )


def _sc_prop(src_p, dst_p, ew_p, hw):
    return _sc_prop_kernel()(src_p, dst_p, ew_p, hw)


def _sc_prop_body(src_hbm, dst_hbm, ew_hbm, hw_hbm, out_hbm,
                  src_v, dst_v, ew_v, rows_a, rows_b, zbuf_v, shared,
                  gsem_a, gsem_b, ssem_a, ssem_b):
    cid = lax.axis_index("c")
    sid = lax.axis_index("s")
    wid = cid * 16 + sid
    _zero_shared_slice(shared, zbuf_v, sid)
    pltpu.sync_copy(src_hbm.at[wid], src_v)
    pltpu.sync_copy(dst_hbm.at[wid], dst_v)
    pltpu.sync_copy(ew_hbm.at[wid], ew_v)
    plsc.subcore_barrier()

    bufs = ((rows_a, gsem_a, ssem_a), (rows_b, gsem_b, ssem_b))

    def _scale(rows_v, j):
        for g in range(CH // 16):
            ew16 = ew_v[j, pl.ds(g * 16, 16)]
            for k in range(16):
                i = g * 16 + k
                rows_v[i] = rows_v[i] * _splat(ew16, k)

    def _gather_start(b, j):
        rows_v, gsem, _ = bufs[b]
        pltpu.async_copy(hw_hbm.at[src_v.at[j]], rows_v, gsem)

    def _process(b, j):
        rows_v, gsem, ssem = bufs[b]
        pltpu.make_async_copy(hw_hbm.at[src_v.at[j]], rows_v, gsem).wait()
        _scale(rows_v, j)
        pltpu.async_copy(rows_v, shared.at[dst_v.at[j]], ssem, add=True)

    def _refill(b, jnext):
        rows_v, gsem, ssem = bufs[b]
        pltpu.make_async_copy(rows_v, shared.at[dst_v.at[jnext]], ssem).wait()
        _gather_start(b, jnext)

    _gather_start(0, 0)
    _gather_start(1, 1)

    @pl.loop(0, NCH // 2 - 1)
    def _(p):
        j = p * 2
        _process(0, j)
        _refill(0, j + 2)
        _process(1, j + 1)
        _refill(1, j + 3)

    _process(0, NCH - 2)
    _process(1, NCH - 1)
    pltpu.make_async_copy(rows_a, shared.at[dst_v.at[NCH - 2]], ssem_a).wait()
    pltpu.make_async_copy(rows_b, shared.at[dst_v.at[NCH - 1]], ssem_b).wait()

    plsc.subcore_barrier()
    _write_out_slice(shared, out_hbm, cid, sid)


_NP = N_PAD // 8
_NTP = 160


def _xw0_body(x_ref, w_ref, out_ref):
    out_ref[...] = jnp.dot(x_ref[...], w_ref[...],
                           preferred_element_type=_F32)


def _xw0(x_r, W0b):
    return pl.pallas_call(
        _xw0_body,
        grid=(_NP // _NTP,),
        in_specs=[
            pl.BlockSpec((_NTP, 8 * D), lambda i: (i, 0)),
            pl.BlockSpec((8 * D, 128), lambda i: (0, 0)),
        ],
        out_specs=pl.BlockSpec((_NTP, 128), lambda i: (i, 0)),
        out_shape=jax.ShapeDtypeStruct((_NP, 128), _F32),
    )(x_r, W0b)


def _layer0_body(d0_ref, d1_ref, xw_ref, dinv_ref, hw_ref):
    deg = d0_ref[...] + d1_ref[...] + 1.0
    dinv = lax.rsqrt(deg)
    dinv_ref[...] = dinv
    hw_ref[...] = dinv * xw_ref[...]


def _layer0(deg0, deg1, xw):
    return pl.pallas_call(
        _layer0_body,
        grid=(_NP // _NTP,),
        in_specs=[
            pl.BlockSpec((_NTP, 128), lambda i: (i, 0)),
            pl.BlockSpec((_NTP, 128), lambda i: (i, 0)),
            pl.BlockSpec((_NTP, 128), lambda i: (i, 0)),
        ],
        out_specs=[
            pl.BlockSpec((_NTP, 128), lambda i: (i, 0)),
            pl.BlockSpec((_NTP, 128), lambda i: (i, 0)),
        ],
        out_shape=[
            jax.ShapeDtypeStruct((_NP, 128), _F32),
            jax.ShapeDtypeStruct((_NP, 128), _F32),
        ],
    )(deg0, deg1, xw)


def _layer_body(a0_ref, a1_ref, hwp_ref, dinv_ref, b_ref, w_ref, out_ref):
    dinv = dinv_ref[...]
    h = jnp.maximum(
        dinv * (a0_ref[...] + a1_ref[...] + hwp_ref[...]) + b_ref[...], 0.0)
    out_ref[...] = dinv * jnp.dot(h, w_ref[...], preferred_element_type=_F32)


def _layer(agg0, agg1, hwp, dinv, bt, Wb):
    return pl.pallas_call(
        _layer_body,
        grid=(_NP // _NTP,),
        in_specs=[
            pl.BlockSpec((_NTP, 128), lambda i: (i, 0)),
            pl.BlockSpec((_NTP, 128), lambda i: (i, 0)),
            pl.BlockSpec((_NTP, 128), lambda i: (i, 0)),
            pl.BlockSpec((_NTP, 128), lambda i: (i, 0)),
            pl.BlockSpec((128,), lambda i: (0,)),
            pl.BlockSpec((128, 128), lambda i: (0, 0)),
        ],
        out_specs=pl.BlockSpec((_NTP, 128), lambda i: (i, 0)),
        out_shape=jax.ShapeDtypeStruct((_NP, 128), _F32),
    )(agg0, agg1, hwp, dinv, bt, Wb)


_BN_SCALE = np.float32(1.0 / np.sqrt(1.0 + 1e-5))


def _head_body(a0_ref, a1_ref, hwp_ref, dinv_ref, b_ref, wc1_ref, bc1_ref,
               g_ref, beta_ref, wc2_ref, bc2_ref, out_ref):
    dinv = dinv_ref[...]
    h = jnp.maximum(
        dinv * (a0_ref[...] + a1_ref[...] + hwp_ref[...]) + b_ref[...], 0.0)
    z = jnp.maximum(
        jnp.dot(h, wc1_ref[...], preferred_element_type=_F32) + bc1_ref[...],
        0.0)
    zbn = z * (g_ref[...] * _BN_SCALE) + beta_ref[...]
    out_ref[...] = jnp.dot(zbn, wc2_ref[...],
                           preferred_element_type=_F32) + bc2_ref[...]


def _head(agg0, agg1, hwp, dinv, b2t, Wc1b, bc1t, gammat, betat, Wc2b, bc2t):
    return pl.pallas_call(
        _head_body,
        grid=(_NP // _NTP,),
        in_specs=[
            pl.BlockSpec((_NTP, 128), lambda i: (i, 0)),
            pl.BlockSpec((_NTP, 128), lambda i: (i, 0)),
            pl.BlockSpec((_NTP, 128), lambda i: (i, 0)),
            pl.BlockSpec((_NTP, 128), lambda i: (i, 0)),
            pl.BlockSpec((128,), lambda i: (0,)),
            pl.BlockSpec((128, 8 * 256), lambda i: (0, 0)),
            pl.BlockSpec((8 * 256,), lambda i: (0,)),
            pl.BlockSpec((8 * 256,), lambda i: (0,)),
            pl.BlockSpec((8 * 256,), lambda i: (0,)),
            pl.BlockSpec((8 * 256, 8 * NC), lambda i: (0, 0)),
            pl.BlockSpec((8 * NC,), lambda i: (0,)),
        ],
        out_specs=pl.BlockSpec((_NTP, 8 * NC), lambda i: (i, 0)),
        out_shape=jax.ShapeDtypeStruct((_NP, 8 * NC), _F32),
    )(agg0, agg1, hwp, dinv, b2t, Wc1b, bc1t, gammat, betat, Wc2b, bc2t)


def kernel(x, edge_index, edgenet_input, We1, be1, W0, b0, W1, b1, W2, b2,
           Wc1, bc1, gamma, beta, Wc2, bc2):
    src = edge_index[0].astype(_I32)
    dst = edge_index[1].astype(_I32)

    ew2d = _edge_mlp(edgenet_input, We1, be1).reshape(E // 128, 128)

    pad = E_PAD - E
    fill = (jnp.arange(pad, dtype=_I32) * 37) % N
    src_p = jnp.concatenate([src, fill]).reshape(NUM_TILES, NCH, CH)
    dst_p = jnp.concatenate([dst, fill]).reshape(NUM_TILES, NCH, CH)
    ew_p = jnp.concatenate([ew2d, jnp.zeros((pad // 128, 128), _F32)]
                           ).reshape(NUM_TILES, NCH, CH)

    x_pad = jnp.concatenate([x, jnp.zeros((N_PAD - N, D), _F32)])
    eye8 = jnp.eye(8, dtype=_F32)
    xw = _xw0(x_pad.reshape(N_PAD // 8, 8 * D), jnp.kron(eye8, W0))

    W1b = jnp.kron(eye8, W1)
    W2b = jnp.kron(eye8, W2)
    Wc1b = jnp.kron(eye8, Wc1)
    Wc2b = jnp.kron(eye8, Wc2)
    b0t = jnp.tile(b0, 8)
    b1t = jnp.tile(b1, 8)
    b2t = jnp.tile(b2, 8)
    bc1t = jnp.tile(bc1, 8)
    gammat = jnp.tile(gamma, 8)
    betat = jnp.tile(beta, 8)
    bc2t = jnp.tile(bc2, 8)

    deg = _sc_deg(dst_p, ew_p)
    degp = deg.reshape(2, N_PAD // 8, 128)
    dinv, hw = _layer0(degp[0], degp[1], xw)

    agg = _sc_prop(src_p, dst_p, ew_p, hw.reshape(N_PAD, 16))
    aggp = agg.reshape(2, N_PAD // 8, 128)
    hw1 = _layer(aggp[0], aggp[1], hw, dinv, b0t, W1b)
    agg = _sc_prop(src_p, dst_p, ew_p, hw1.reshape(N_PAD, 16))
    aggp = agg.reshape(2, N_PAD // 8, 128)
    hw2 = _layer(aggp[0], aggp[1], hw1, dinv, b1t, W2b)
    agg = _sc_prop(src_p, dst_p, ew_p, hw2.reshape(N_PAD, 16))
    aggp = agg.reshape(2, N_PAD // 8, 128)

    logit = _head(aggp[0], aggp[1], hw2, dinv, b2t, Wc1b, bc1t, gammat,
                  betat, Wc2b, bc2t)
    return logit.reshape(N_PAD, NC)[:N]
